# Initial kernel scaffold; baseline (speedup 1.0000x reference)
import jax
import jax.numpy as jnp
from jax.experimental import pallas as pl


def kernel(x, W1, g1, b1, W2, g2, b2, W3, g3, b3, W4, g4, b4, W5, g5, b5, Wl1, g6, b6, Wl2, bl2, g7, b7, Wl3, bl3):
    raise NotImplementedError("write your pallas kernel here")


# speedup vs baseline: 10.0817x; 10.0817x over previous
import functools

import jax
import jax.numpy as jnp
from jax import lax
from jax.experimental import pallas as pl
from jax.experimental.pallas import tpu as pltpu
from jax.experimental.pallas import tpu_sc as plsc

B = 8
N = 2048
K = 20
EPS = 1e-5
BN = B * N

NC = 2
NS = 16
NW = NC * NS
P = BN // NW
G_PTS = 4
GK = G_PTS * K
NG = P // G_PTS

_HIGH = lax.Precision.HIGHEST
_IMIN = jnp.iinfo(jnp.int32).min


def _lrelu(v):
    return jnp.maximum(v, 0.2 * v)


@functools.lru_cache(maxsize=None)
def _make_knn(C, R=256):
    def body(xr_ref, xcn_ref, idx_ref):
        b = pl.program_id(0)
        xr = xr_ref[0]
        xcn = xcn_ref[0]
        xxm = jnp.sum(xcn * xcn, axis=0, keepdims=True)
        ip = lax.dot_general(xr, xcn, (((1,), (0,)), ((), ())),
                             preferred_element_type=jnp.float32)
        z = 2.0 * ip - xxm
        bits = lax.bitcast_convert_type(z, jnp.int32)
        skey = bits ^ (lax.shift_right_arithmetic(bits, 31)
                       & jnp.int32(0x7FFFFFFF))
        lane = lax.broadcasted_iota(jnp.int32, (R, N), 1)
        base = b * N
        cols = []
        for _ in range(K):
            m = jnp.max(skey, axis=1, keepdims=True)
            sel = jnp.where(skey == m, lane, jnp.int32(N))
            l = jnp.min(sel, axis=1, keepdims=True)
            cols.append(l + base)
            skey = jnp.where(lane == l, _IMIN, skey)
        idx_ref[0] = jnp.concatenate(cols, axis=1)

    return pl.pallas_call(
        body,
        grid=(B, N // R),
        in_specs=[pl.BlockSpec((1, R, C), lambda b, r: (b, r, 0)),
                  pl.BlockSpec((1, C, N), lambda b, r: (b, 0, 0))],
        out_specs=pl.BlockSpec((1, R, K), lambda b, r: (b, r, 0)),
        out_shape=jax.ShapeDtypeStruct((B, N, K), jnp.int32),
    )


@functools.lru_cache(maxsize=None)
def _make_gc(C, O):
    def body(xt_ref, w_ref, g_ref, b_ref, go_ref, co_ref):
        xt0 = xt_ref[0]
        w = w_ref[...]
        wd = w[:, :C]
        wc = w[:, C:]
        s = g_ref[...] / jnp.sqrt(jnp.float32(1.0 + EPS))
        gm = lax.dot_general(xt0, wd, (((1,), (1,)), ((), ())),
                             preferred_element_type=jnp.float32)
        cm = lax.dot_general(xt0, wc - wd, (((1,), (1,)), ((), ())),
                             preferred_element_type=jnp.float32)
        go_ref[0] = gm * s
        co_ref[0] = cm * s + b_ref[...]

    return pl.pallas_call(
        body,
        grid=(B,),
        in_specs=[pl.BlockSpec((1, N, C), lambda b: (b, 0, 0)),
                  pl.BlockSpec((O, 2 * C), lambda b: (0, 0)),
                  pl.BlockSpec((1, O), lambda b: (0, 0)),
                  pl.BlockSpec((1, O), lambda b: (0, 0))],
        out_specs=[pl.BlockSpec((1, N, O), lambda b: (b, 0, 0)),
                   pl.BlockSpec((1, N, O), lambda b: (b, 0, 0))],
        out_shape=[jax.ShapeDtypeStruct((B, N, O), jnp.float32),
                   jax.ShapeDtypeStruct((B, N, O), jnp.float32)],
    )


@functools.lru_cache(maxsize=None)
def _make_scgm(O):
    @functools.partial(
        pl.kernel,
        mesh=plsc.VectorSubcoreMesh(core_axis_name="c", subcore_axis_name="s"),
        compiler_params=pltpu.CompilerParams(use_tc_tiling_on_sc=False),
        out_type=jax.ShapeDtypeStruct((BN, O), jnp.float32),
        scratch_types=[
            pltpu.VMEM((NG, GK), jnp.int32),
            pltpu.VMEM((GK, O), jnp.float32),
            pltpu.VMEM((GK, O), jnp.float32),
            pltpu.VMEM((G_PTS, O), jnp.float32),
            pltpu.VMEM((G_PTS, O), jnp.float32),
            pltpu.VMEM((G_PTS, O), jnp.float32),
            pltpu.VMEM((G_PTS, O), jnp.float32),
            pltpu.SemaphoreType.DMA,
            pltpu.SemaphoreType.DMA,
            pltpu.SemaphoreType.DMA,
            pltpu.SemaphoreType.DMA,
        ],
    )
    def k(tab_hbm, idx_hbm, cent_hbm, out_hbm,
          idx_v, rows0, rows1, cb0, cb1, ob0, ob1, s0, s1, s2, s3):
        wid = lax.axis_index("s") * NC + lax.axis_index("c")
        base_pt = wid * P
        pltpu.sync_copy(idx_hbm.at[wid], idx_v)
        rows = (rows0, rows1)
        cbs = (cb0, cb1)
        obs = (ob0, ob1)
        sems = ((s0, s2), (s1, s3))

        def outer(i, _):
            g0 = i * 2
            copies = []
            for bf in (0, 1):
                g = g0 + bf
                c1 = pltpu.async_copy(tab_hbm.at[idx_v.at[g]],
                                      rows[bf], sems[bf][0])
                c2 = pltpu.async_copy(
                    cent_hbm.at[pl.ds(base_pt + g * G_PTS, G_PTS)],
                    cbs[bf], sems[bf][1])
                copies.append((c1, c2))
            for bf in (0, 1):
                g = g0 + bf
                copies[bf][0].wait()
                copies[bf][1].wait()
                for p in range(G_PTS):
                    def jbody(j, _, _p=p, _bf=bf):
                        off = pl.multiple_of(j * 16, 16)
                        acc = rows[_bf][_p * K, pl.ds(off, 16)]
                        for kk in range(1, K):
                            acc = jnp.maximum(
                                acc, rows[_bf][_p * K + kk, pl.ds(off, 16)])
                        val = acc + cbs[_bf][_p, pl.ds(off, 16)]
                        obs[_bf][_p, pl.ds(off, 16)] = _lrelu(val)
                        return 0
                    lax.fori_loop(0, O // 16, jbody, 0)
                pltpu.sync_copy(
                    obs[bf], out_hbm.at[pl.ds(base_pt + g * G_PTS, G_PTS)])
            return 0

        lax.fori_loop(0, NG // 2, outer, 0)

    return k


def _make_pool(R=256):
    def body(x1_ref, x2_ref, x3_ref, x4_ref, w_ref, g_ref, b_ref,
             p1_ref, ps_ref):
        r = pl.program_id(1)
        xc = jnp.concatenate(
            [x1_ref[0], x2_ref[0], x3_ref[0], x4_ref[0]], axis=1)
        y = lax.dot_general(xc, w_ref[...], (((1,), (1,)), ((), ())),
                            preferred_element_type=jnp.float32)
        s = g_ref[...] / jnp.sqrt(jnp.float32(1.0 + EPS))
        y = _lrelu(y * s + b_ref[...])
        mx = jnp.max(y, axis=0, keepdims=True)
        sm = jnp.sum(y, axis=0, keepdims=True)

        @pl.when(r == 0)
        def _():
            p1_ref[0] = mx
            ps_ref[0] = sm

        @pl.when(r != 0)
        def _():
            p1_ref[0] = jnp.maximum(p1_ref[0], mx)
            ps_ref[0] = ps_ref[0] + sm

    return pl.pallas_call(
        body,
        grid=(B, N // R),
        in_specs=[pl.BlockSpec((1, R, 64), lambda b, r: (b, r, 0)),
                  pl.BlockSpec((1, R, 64), lambda b, r: (b, r, 0)),
                  pl.BlockSpec((1, R, 128), lambda b, r: (b, r, 0)),
                  pl.BlockSpec((1, R, 256), lambda b, r: (b, r, 0)),
                  pl.BlockSpec((1024, 512), lambda b, r: (0, 0)),
                  pl.BlockSpec((1, 1024), lambda b, r: (0, 0)),
                  pl.BlockSpec((1, 1024), lambda b, r: (0, 0))],
        out_specs=[pl.BlockSpec((1, 1, 1024), lambda b, r: (b, 0, 0)),
                   pl.BlockSpec((1, 1, 1024), lambda b, r: (b, 0, 0))],
        out_shape=[jax.ShapeDtypeStruct((B, 1, 1024), jnp.float32),
                   jax.ShapeDtypeStruct((B, 1, 1024), jnp.float32)],
    )


def _make_mlp():
    def body(p1_ref, ps_ref, w1_ref, g6_ref, b6_ref, w2_ref, bl2_ref,
             g7_ref, b7_ref, w3_ref, bl3_ref, out_ref):
        inv = jnp.float32(1.0 / N)
        z = jnp.concatenate([p1_ref[...], ps_ref[...] * inv], axis=1)
        rs = jnp.sqrt(jnp.float32(1.0 + EPS))
        h = lax.dot_general(z, w1_ref[...], (((1,), (1,)), ((), ())),
                            preferred_element_type=jnp.float32)
        h = _lrelu(h * (g6_ref[...] / rs) + b6_ref[...])
        h = lax.dot_general(h, w2_ref[...], (((1,), (1,)), ((), ())),
                            preferred_element_type=jnp.float32) + bl2_ref[...]
        h = _lrelu(h * (g7_ref[...] / rs) + b7_ref[...])
        o = lax.dot_general(h, w3_ref[...], (((1,), (1,)), ((), ())),
                            preferred_element_type=jnp.float32) + bl3_ref[...]
        out_ref[...] = o

    return pl.pallas_call(
        body,
        out_shape=jax.ShapeDtypeStruct((B, 40), jnp.float32),
    )


def kernel(x, W1, g1, b1, W2, g2, b2, W3, g3, b3, W4, g4, b4,
           W5, g5, b5, Wl1, g6, b6, Wl2, bl2, g7, b7, Wl3, bl3):
    xt = jnp.transpose(x, (0, 2, 1))
    feats = []
    cur = xt
    cur_cn = x
    for (W, g, b) in ((W1, g1, b1), (W2, g2, b2), (W3, g3, b3),
                      (W4, g4, b4)):
        O, twoC = W.shape
        C = twoC // 2
        idx = _make_knn(C)(cur, cur_cn)
        gm, cm = _make_gc(C, O)(cur, W, g.reshape(1, O), b.reshape(1, O))
        out = _make_scgm(O)(gm.reshape(BN, O),
                            idx.reshape(NW, NG, GK),
                            cm.reshape(BN, O))
        cur = out.reshape(B, N, O)
        cur_cn = jnp.transpose(cur, (0, 2, 1))
        feats.append(cur)
    p1, ps = _make_pool()(feats[0], feats[1], feats[2], feats[3],
                          W5, g5.reshape(1, -1), b5.reshape(1, -1))
    p1 = p1.reshape(B, 1024)
    ps = ps.reshape(B, 1024)
    z = _make_mlp()(p1, ps, Wl1, g6.reshape(1, -1), b6.reshape(1, -1),
                    Wl2, bl2.reshape(1, -1), g7.reshape(1, -1),
                    b7.reshape(1, -1), Wl3, bl3.reshape(1, -1))
    return z
